# baseline (speedup 1.0000x reference)
import jax
import jax.numpy as jnp
from jax.experimental import pallas as pl
from jax.experimental.pallas import tpu as pltpu

_M = 32
_CIN = 3
_COUT = 20
_NBLK = 2
_G = 8
_RTILE = 512


def _mlp_kernel(x_ref, w1_ref, wr_ref, bn_ref, w2_ref, blin_ref, o_ref):
    bn = bn_ref[...]
    g = [bn[k] for k in range(8)]

    x = x_ref[...].astype(jnp.bfloat16)
    h = jnp.dot(x, w1_ref[...], preferred_element_type=jnp.float32)

    for b in range(_NBLK):
        t = jnp.maximum(h * g[b] + g[3 + b], 0.0)
        t = jnp.dot(t.astype(jnp.bfloat16), wr_ref[2 * b],
                    preferred_element_type=jnp.float32)
        t = jnp.maximum(t + g[5 + b], 0.0)
        t = jnp.dot(t.astype(jnp.bfloat16), wr_ref[2 * b + 1],
                    preferred_element_type=jnp.float32)
        h = h + t

    h = jnp.maximum(h * g[2] + g[7], 0.0)
    out = jnp.dot(h.astype(jnp.bfloat16), w2_ref[...],
                  preferred_element_type=jnp.float32) + blin_ref[...]
    o_ref[...] = out.astype(jnp.bfloat16)


def _block_diag(w):
    eye = jnp.eye(_G, dtype=w.dtype)
    i, o = w.shape
    return (eye[:, None, :, None] * w[None, :, None, :]).reshape(_G * i, _G * o)


def _pack_params(wsub, wres, gres, bres, gfin, bfin, wlin, blin):
    bf = jnp.bfloat16
    w1 = _block_diag(wsub).astype(bf)
    wr = []
    for b in range(_NBLK):
        wr.append(_block_diag(wres[2 * b] * gres[2 * b + 1].reshape(1, _M)))
        wr.append(_block_diag(wres[2 * b + 1]))
    wr = jnp.stack(wr).astype(bf)
    w2 = _block_diag(wlin).astype(bf)

    tile = lambda v: jnp.tile(v.reshape(-1).astype(jnp.float32), (_G,))
    bn = jnp.stack([
        tile(gres[0]), tile(gres[2]), tile(gfin),
        tile(bres[0]), tile(bres[2]),
        tile(bres[1]), tile(bres[3]), tile(bfin),
    ])
    blin_big = jnp.tile(blin.reshape(-1).astype(jnp.float32),
                        (_G,)).reshape(1, _G * _COUT)
    return w1, wr, bn, w2, blin_big


def kernel(x, wsub, wres, gres, bres, gfin, bfin, wlin, blin):
    n = x.shape[0]
    pts_per_tile = _G * _RTILE
    n_pad = pl.cdiv(n, pts_per_tile) * pts_per_tile
    if n_pad != n:
        x = jnp.pad(x, ((0, n_pad - n), (0, 0)))
    x2 = x.reshape(n_pad // _G, _G * _CIN)

    w1, wr, bn, w2, blin_big = _pack_params(
        wsub, wres, gres, bres, gfin, bfin, wlin, blin)

    full = lambda shape: pl.BlockSpec(shape, lambda i: (0,) * len(shape))
    out2 = pl.pallas_call(
        _mlp_kernel,
        out_shape=jax.ShapeDtypeStruct((n_pad // _G, _G * _COUT), jnp.bfloat16),
        grid=(n_pad // pts_per_tile,),
        in_specs=[
            pl.BlockSpec((_RTILE, _G * _CIN), lambda i: (i, 0)),
            full((_G * _CIN, _G * _M)),
            full((2 * _NBLK, _G * _M, _G * _M)),
            full((8, _G * _M)),
            full((_G * _M, _G * _COUT)),
            full((1, _G * _COUT)),
        ],
        out_specs=pl.BlockSpec((_RTILE, _G * _COUT), lambda i: (i, 0)),
        compiler_params=pltpu.CompilerParams(
            dimension_semantics=("parallel",)),
    )(x2, w1, wr, bn, w2, blin_big)

    out = out2.reshape(n_pad, _COUT)
    return out if n_pad == n else out[:n]


# speedup vs baseline: 1.0207x; 1.0207x over previous
import jax
import jax.numpy as jnp
from jax.experimental import pallas as pl
from jax.experimental.pallas import tpu as pltpu

_M = 32
_CIN = 3
_COUT = 20
_NBLK = 2
_SUB = 512
_TILE = 4096


def _unet_kernel(x_ref, wsub_ref, wres_ref, bn_ref, wlin_ref, blin_ref, o_ref):
    wsub = wsub_ref[...]
    wres = [wres_ref[k] for k in range(2 * _NBLK)]
    wlin = wlin_ref[...]
    blin = blin_ref[...]
    bn = bn_ref[...]
    g = [bn[:, k:k + 1] for k in range(8)]

    for j in range(_TILE // _SUB):
        sl = pl.ds(j * _SUB, _SUB)
        xc = x_ref[sl, :].astype(jnp.bfloat16)
        hn = jnp.dot(xc, wsub, preferred_element_type=jnp.float32)
        h = hn.T

        for b in range(_NBLK):
            t = jnp.maximum(h * g[b] + g[3 + b], 0.0)
            t = jnp.dot(wres[2 * b], t.astype(jnp.bfloat16),
                        preferred_element_type=jnp.float32)
            t = jnp.maximum(t + g[5 + b], 0.0)
            t = jnp.dot(wres[2 * b + 1], t.astype(jnp.bfloat16),
                        preferred_element_type=jnp.float32)
            h = h + t

        h = jnp.maximum(h * g[2] + g[7], 0.0)
        out = jnp.dot(wlin, h.astype(jnp.bfloat16),
                      preferred_element_type=jnp.float32) + blin
        on = out.astype(jnp.bfloat16).T
        o_ref[sl, :] = on[:, :_COUT]


def _pack_params(wsub, wres, gres, bres, gfin, bfin, wlin, blin):
    bf = jnp.bfloat16
    wsub_p = wsub.astype(bf)
    wres_p = []
    for b in range(_NBLK):
        g1 = gres[2 * b + 1].reshape(_M, 1)
        wres_p.append((wres[2 * b].T * g1).astype(bf))
        wres_p.append(wres[2 * b + 1].T.astype(bf))
    wres_p = jnp.stack(wres_p)

    wlin_p = jnp.zeros((_M, _M), jnp.float32).at[:_COUT].set(wlin.T).astype(bf)
    blin_p = jnp.zeros((_M, 1), jnp.float32).at[:_COUT].set(blin.reshape(_COUT, 1))

    col = lambda v: v.reshape(-1).astype(jnp.float32)
    bn = jnp.stack([
        col(gres[0]), col(gres[2]), col(gfin),
        col(bres[0]), col(bres[2]),
        col(bres[1]), col(bres[3]), col(bfin),
    ], axis=1)
    return wsub_p, wres_p, bn, wlin_p, blin_p


def kernel(x, wsub, wres, gres, bres, gfin, bfin, wlin, blin):
    n = x.shape[0]
    n_pad = pl.cdiv(n, _TILE) * _TILE
    if n_pad != n:
        x = jnp.pad(x, ((0, n_pad - n), (0, 0)))

    wsub_p, wres_p, bn, wlin_p, blin_p = _pack_params(
        wsub, wres, gres, bres, gfin, bfin, wlin, blin)

    full = lambda shape: pl.BlockSpec(shape, lambda i: (0,) * len(shape))
    out = pl.pallas_call(
        _unet_kernel,
        out_shape=jax.ShapeDtypeStruct((n_pad, _COUT), jnp.bfloat16),
        grid=(n_pad // _TILE,),
        in_specs=[
            pl.BlockSpec((_TILE, _CIN), lambda i: (i, 0)),
            full((_CIN, _M)),
            full((2 * _NBLK, _M, _M)),
            full((_M, 8)),
            full((_M, _M)),
            full((_M, 1)),
        ],
        out_specs=pl.BlockSpec((_TILE, _COUT), lambda i: (i, 0)),
        compiler_params=pltpu.CompilerParams(
            dimension_semantics=("parallel",)),
    )(x, wsub_p, wres_p, bn, wlin_p, blin_p)

    return out if n_pad == n else out[:n]


# speedup vs baseline: 5.6541x; 5.5393x over previous
import jax
import jax.numpy as jnp
from jax.experimental import pallas as pl
from jax.experimental.pallas import tpu as pltpu

_M = 32
_CIN = 3
_COUT = 20
_NBLK = 2
_G = 8
_SUB = 512
_TILE = _G * _SUB


def _unet_kernel(x_ref, w1_ref, wr_ref, bn_ref, w2_ref, o_ref):
    xt = x_ref[...]
    x8 = jnp.concatenate(
        [xt[:, _SUB * k:_SUB * (k + 1)] for k in range(_G)], axis=0)

    bn = bn_ref[...]
    g = [bn[:, k:k + 1] for k in range(9)]

    h = jnp.dot(w1_ref[...], x8, preferred_element_type=jnp.float32)

    for b in range(_NBLK):
        t = jnp.maximum(h * g[b] + g[3 + b], 0.0)
        t = jnp.dot(wr_ref[2 * b], t.astype(jnp.bfloat16),
                    preferred_element_type=jnp.float32)
        t = jnp.maximum(t + g[5 + b], 0.0)
        t = jnp.dot(wr_ref[2 * b + 1], t.astype(jnp.bfloat16),
                    preferred_element_type=jnp.float32)
        h = h + t

    h = jnp.maximum(h * g[2] + g[7], 0.0)
    out = jnp.dot(w2_ref[...], h.astype(jnp.bfloat16),
                  preferred_element_type=jnp.float32) + g[8]
    ob = out.astype(jnp.bfloat16)
    for k in range(_G):
        o_ref[:, pl.ds(_SUB * k, _SUB)] = ob[_M * k:_M * k + _COUT, :]


def _expand(w, rows, cols):
    eye = jnp.eye(_G, dtype=w.dtype)
    r, c = w.shape
    out = (eye[:, None, :, None] * w[None, :, None, :])
    return out.reshape(_G * r, _G * c).astype(jnp.bfloat16)


def _pack_params(wsub, wres, gres, bres, gfin, bfin, wlin, blin):
    w1 = _expand(wsub.T, _M, _CIN)
    wr = []
    for b in range(_NBLK):
        g1 = gres[2 * b + 1].reshape(_M, 1)
        wr.append(_expand(wres[2 * b].T * g1, _M, _M))
        wr.append(_expand(wres[2 * b + 1].T, _M, _M))
    wr = jnp.stack(wr)

    wlin_pad = jnp.zeros((_M, _M), jnp.float32).at[:_COUT].set(wlin.T)
    w2 = _expand(wlin_pad, _M, _M)

    tile8 = lambda v: jnp.tile(v.reshape(-1).astype(jnp.float32), (_G,))
    blin_pad = jnp.zeros((_M,), jnp.float32).at[:_COUT].set(blin.reshape(-1))
    bn = jnp.stack([
        tile8(gres[0]), tile8(gres[2]), tile8(gfin),
        tile8(bres[0]), tile8(bres[2]),
        tile8(bres[1]), tile8(bres[3]), tile8(bfin),
        tile8(blin_pad),
    ], axis=1)
    return w1, wr, bn, w2


def kernel(x, wsub, wres, gres, bres, gfin, bfin, wlin, blin):
    n = x.shape[0]
    n_pad = pl.cdiv(n, _TILE) * _TILE
    x_t = jnp.pad(x.astype(jnp.bfloat16), ((0, n_pad - n), (0, 0))).T

    w1, wr, bn, w2 = _pack_params(
        wsub, wres, gres, bres, gfin, bfin, wlin, blin)

    full = lambda shape: pl.BlockSpec(shape, lambda i: (0,) * len(shape))
    out_t = pl.pallas_call(
        _unet_kernel,
        out_shape=jax.ShapeDtypeStruct((_COUT, n_pad), jnp.bfloat16),
        grid=(n_pad // _TILE,),
        in_specs=[
            pl.BlockSpec((_CIN, _TILE), lambda i: (0, i)),
            full((_G * _M, _G * _CIN)),
            full((2 * _NBLK, _G * _M, _G * _M)),
            full((_G * _M, 9)),
            full((_G * _M, _G * _M)),
        ],
        out_specs=pl.BlockSpec((_COUT, _TILE), lambda i: (0, i)),
        compiler_params=pltpu.CompilerParams(
            dimension_semantics=("parallel",)),
    )(x_t, w1, wr, bn, w2)

    return out_t[:, :n].T


# speedup vs baseline: 9.8289x; 1.7384x over previous
import jax
import jax.numpy as jnp
from jax.experimental import pallas as pl
from jax.experimental.pallas import tpu as pltpu

_M = 32
_CIN = 3
_COUT = 20
_NBLK = 2
_G = 8
_SUB = 1024
_TILE = _G * _SUB


def _unet_kernel(x_ref, w1_ref, wr_ref, bn_ref, w2_ref, o_ref):
    xt = x_ref[...]
    x8 = jnp.concatenate(
        [xt[:, _SUB * k:_SUB * (k + 1)] for k in range(_G)], axis=0)

    bn = bn_ref[...]
    g = [bn[:, k:k + 1] for k in range(9)]

    h = jnp.dot(w1_ref[...], x8, preferred_element_type=jnp.float32)

    for b in range(_NBLK):
        t = jnp.maximum(h * g[b] + g[3 + b], 0.0)
        t = jnp.dot(wr_ref[2 * b], t.astype(jnp.bfloat16),
                    preferred_element_type=jnp.float32)
        t = jnp.maximum(t + g[5 + b], 0.0)
        t = jnp.dot(wr_ref[2 * b + 1], t.astype(jnp.bfloat16),
                    preferred_element_type=jnp.float32)
        h = h + t

    h = jnp.maximum(h * g[2] + g[7], 0.0)
    out = jnp.dot(w2_ref[...], h.astype(jnp.bfloat16),
                  preferred_element_type=jnp.float32) + g[8]
    ob = out.astype(jnp.bfloat16)
    for k in range(_G):
        o_ref[:, pl.ds(_SUB * k, _SUB)] = ob[_M * k:_M * k + _COUT, :]


def _expand(w, rows, cols):
    eye = jnp.eye(_G, dtype=w.dtype)
    r, c = w.shape
    out = (eye[:, None, :, None] * w[None, :, None, :])
    return out.reshape(_G * r, _G * c).astype(jnp.bfloat16)


def _pack_params(wsub, wres, gres, bres, gfin, bfin, wlin, blin):
    w1 = _expand(wsub.T, _M, _CIN)
    wr = []
    for b in range(_NBLK):
        g1 = gres[2 * b + 1].reshape(_M, 1)
        wr.append(_expand(wres[2 * b].T * g1, _M, _M))
        wr.append(_expand(wres[2 * b + 1].T, _M, _M))
    wr = jnp.stack(wr)

    wlin_pad = jnp.zeros((_M, _M), jnp.float32).at[:_COUT].set(wlin.T)
    w2 = _expand(wlin_pad, _M, _M)

    tile8 = lambda v: jnp.tile(v.reshape(-1).astype(jnp.float32), (_G,))
    blin_pad = jnp.zeros((_M,), jnp.float32).at[:_COUT].set(blin.reshape(-1))
    bn = jnp.stack([
        tile8(gres[0]), tile8(gres[2]), tile8(gfin),
        tile8(bres[0]), tile8(bres[2]),
        tile8(bres[1]), tile8(bres[3]), tile8(bfin),
        tile8(blin_pad),
    ], axis=1)
    return w1, wr, bn, w2


def kernel(x, wsub, wres, gres, bres, gfin, bfin, wlin, blin):
    n = x.shape[0]
    n_pad = pl.cdiv(n, _TILE) * _TILE
    x_t = jnp.pad(x.astype(jnp.bfloat16), ((0, n_pad - n), (0, 0))).T

    w1, wr, bn, w2 = _pack_params(
        wsub, wres, gres, bres, gfin, bfin, wlin, blin)

    full = lambda shape: pl.BlockSpec(shape, lambda i: (0,) * len(shape))
    out_t = pl.pallas_call(
        _unet_kernel,
        out_shape=jax.ShapeDtypeStruct((_COUT, n_pad), jnp.bfloat16),
        grid=(n_pad // _TILE,),
        in_specs=[
            pl.BlockSpec((_CIN, _TILE), lambda i: (0, i)),
            full((_G * _M, _G * _CIN)),
            full((2 * _NBLK, _G * _M, _G * _M)),
            full((_G * _M, 9)),
            full((_G * _M, _G * _M)),
        ],
        out_specs=pl.BlockSpec((_COUT, _TILE), lambda i: (0, i)),
        compiler_params=pltpu.CompilerParams(
            dimension_semantics=("parallel",)),
    )(x_t, w1, wr, bn, w2)

    return out_t[:, :n].T


# speedup vs baseline: 14.2108x; 1.4458x over previous
import jax
import jax.numpy as jnp
from jax.experimental import pallas as pl
from jax.experimental.pallas import tpu as pltpu

_M = 32
_CIN = 3
_COUT = 20
_NBLK = 2
_G = 8
_SUB = 2048
_TILE = _G * _SUB


def _unet_kernel(x_ref, w1_ref, wr_ref, bn_ref, w2_ref, o_ref):
    xt = x_ref[...]
    x8 = jnp.concatenate(
        [xt[:, _SUB * k:_SUB * (k + 1)] for k in range(_G)], axis=0)

    bn = bn_ref[...]
    g = [bn[:, k:k + 1] for k in range(9)]

    h = jnp.dot(w1_ref[...], x8, preferred_element_type=jnp.float32)

    for b in range(_NBLK):
        t = jnp.maximum(h * g[b] + g[3 + b], 0.0)
        t = jnp.dot(wr_ref[2 * b], t.astype(jnp.bfloat16),
                    preferred_element_type=jnp.float32)
        t = jnp.maximum(t + g[5 + b], 0.0)
        t = jnp.dot(wr_ref[2 * b + 1], t.astype(jnp.bfloat16),
                    preferred_element_type=jnp.float32)
        h = h + t

    h = jnp.maximum(h * g[2] + g[7], 0.0)
    out = jnp.dot(w2_ref[...], h.astype(jnp.bfloat16),
                  preferred_element_type=jnp.float32) + g[8]
    ob = out.astype(jnp.bfloat16)
    for k in range(_G):
        o_ref[:, pl.ds(_SUB * k, _SUB)] = ob[_M * k:_M * k + _COUT, :]


def _expand(w, rows, cols):
    eye = jnp.eye(_G, dtype=w.dtype)
    r, c = w.shape
    out = (eye[:, None, :, None] * w[None, :, None, :])
    return out.reshape(_G * r, _G * c).astype(jnp.bfloat16)


def _pack_params(wsub, wres, gres, bres, gfin, bfin, wlin, blin):
    w1 = _expand(wsub.T, _M, _CIN)
    wr = []
    for b in range(_NBLK):
        g1 = gres[2 * b + 1].reshape(_M, 1)
        wr.append(_expand(wres[2 * b].T * g1, _M, _M))
        wr.append(_expand(wres[2 * b + 1].T, _M, _M))
    wr = jnp.stack(wr)

    wlin_pad = jnp.zeros((_M, _M), jnp.float32).at[:_COUT].set(wlin.T)
    w2 = _expand(wlin_pad, _M, _M)

    tile8 = lambda v: jnp.tile(v.reshape(-1).astype(jnp.float32), (_G,))
    blin_pad = jnp.zeros((_M,), jnp.float32).at[:_COUT].set(blin.reshape(-1))
    bn = jnp.stack([
        tile8(gres[0]), tile8(gres[2]), tile8(gfin),
        tile8(bres[0]), tile8(bres[2]),
        tile8(bres[1]), tile8(bres[3]), tile8(bfin),
        tile8(blin_pad),
    ], axis=1)
    return w1, wr, bn, w2


def kernel(x, wsub, wres, gres, bres, gfin, bfin, wlin, blin):
    n = x.shape[0]
    n_pad = pl.cdiv(n, _TILE) * _TILE
    x_t = jnp.pad(x.astype(jnp.bfloat16), ((0, n_pad - n), (0, 0))).T

    w1, wr, bn, w2 = _pack_params(
        wsub, wres, gres, bres, gfin, bfin, wlin, blin)

    full = lambda shape: pl.BlockSpec(shape, lambda i: (0,) * len(shape))
    out_t = pl.pallas_call(
        _unet_kernel,
        out_shape=jax.ShapeDtypeStruct((_COUT, n_pad), jnp.bfloat16),
        grid=(n_pad // _TILE,),
        in_specs=[
            pl.BlockSpec((_CIN, _TILE), lambda i: (0, i)),
            full((_G * _M, _G * _CIN)),
            full((2 * _NBLK, _G * _M, _G * _M)),
            full((_G * _M, 9)),
            full((_G * _M, _G * _M)),
        ],
        out_specs=pl.BlockSpec((_COUT, _TILE), lambda i: (0, i)),
        compiler_params=pltpu.CompilerParams(
            dimension_semantics=("parallel",)),
    )(x_t, w1, wr, bn, w2)

    return out_t[:, :n].T


# speedup vs baseline: 15.3129x; 1.0776x over previous
import jax
import jax.numpy as jnp
from jax.experimental import pallas as pl
from jax.experimental.pallas import tpu as pltpu

_M = 32
_CIN = 3
_COUT = 20
_NBLK = 2
_G = 8
_SUB = 2048
_TILE = _G * _SUB


def _unet_kernel(x_ref, w1_ref, wr_ref, bn_ref, w2_ref, blin_ref, o_ref):
    xt = x_ref[...]
    x8 = jnp.concatenate(
        [xt[:, _SUB * k:_SUB * (k + 1)] for k in range(_G)], axis=0)

    bn = bn_ref[...]
    g = [bn[:, k:k + 1] for k in range(5)]
    blin = blin_ref[...]

    h = jnp.dot(w1_ref[...], x8, preferred_element_type=jnp.float32)

    for b in range(_NBLK):
        t = jnp.maximum(h + g[b], 0.0)
        t = jnp.dot(wr_ref[2 * b], t.astype(jnp.bfloat16),
                    preferred_element_type=jnp.float32)
        t = jnp.maximum(t + g[2 + b], 0.0)
        t = jnp.dot(wr_ref[2 * b + 1], t.astype(jnp.bfloat16),
                    preferred_element_type=jnp.float32)
        h = h + t

    h = jnp.maximum(h + g[4], 0.0)
    out = jnp.dot(w2_ref[...], h.astype(jnp.bfloat16),
                  preferred_element_type=jnp.float32) + blin
    ob = out.astype(jnp.bfloat16)
    for k in range(_G):
        o_ref[:, pl.ds(_SUB * k, _SUB)] = ob[24 * k:24 * k + _COUT, :]


def _expand(w, rows, cols):
    eye = jnp.eye(_G, dtype=w.dtype)
    r, c = w.shape
    out = (eye[:, None, :, None] * w[None, :, None, :])
    return out.reshape(_G * r, _G * c).astype(jnp.bfloat16)


def _pack_params(wsub, wres, gres, bres, gfin, bfin, wlin, blin):
    w1 = _expand(wsub.T, _M, _CIN)
    wr = []
    for b in range(_NBLK):
        g1 = gres[2 * b + 1].reshape(_M, 1)
        g0 = gres[2 * b].reshape(1, _M)
        wr.append(_expand(wres[2 * b].T * g1 * g0, _M, _M))
        wr.append(_expand(wres[2 * b + 1].T, _M, _M))
    wr = jnp.stack(wr)

    wlin_pad = jnp.zeros((24, _M), jnp.float32).at[:_COUT].set(wlin.T)
    w2 = _expand(wlin_pad * gfin.reshape(1, _M), 24, _M)

    tile8 = lambda v: jnp.tile(v.reshape(-1).astype(jnp.float32), (_G,))
    bn = jnp.stack([
        tile8(bres[0] / gres[0]), tile8(bres[2] / gres[2]),
        tile8(bres[1]), tile8(bres[3]),
        tile8(bfin / gfin),
    ], axis=1)
    blin_pad = jnp.zeros((24,), jnp.float32).at[:_COUT].set(blin.reshape(-1))
    blin_c = tile8(blin_pad).reshape(_G * 24, 1)
    return w1, wr, bn, w2, blin_c


def kernel(x, wsub, wres, gres, bres, gfin, bfin, wlin, blin):
    n = x.shape[0]
    n_pad = pl.cdiv(n, _TILE) * _TILE
    x_t = jnp.pad(x.astype(jnp.bfloat16), ((0, n_pad - n), (0, 0))).T

    w1, wr, bn, w2, blin_c = _pack_params(
        wsub, wres, gres, bres, gfin, bfin, wlin, blin)

    full = lambda shape: pl.BlockSpec(shape, lambda i: (0,) * len(shape))
    out_t = pl.pallas_call(
        _unet_kernel,
        out_shape=jax.ShapeDtypeStruct((_COUT, n_pad), jnp.bfloat16),
        grid=(n_pad // _TILE,),
        in_specs=[
            pl.BlockSpec((_CIN, _TILE), lambda i: (0, i)),
            full((_G * _M, _G * _CIN)),
            full((2 * _NBLK, _G * _M, _G * _M)),
            full((_G * _M, 5)),
            full((_G * 24, _G * _M)),
            full((_G * 24, 1)),
        ],
        out_specs=pl.BlockSpec((_COUT, _TILE), lambda i: (0, i)),
        compiler_params=pltpu.CompilerParams(
            dimension_semantics=("parallel",)),
    )(x_t, w1, wr, bn, w2, blin_c)

    return out_t[:, :n].T


# speedup vs baseline: 15.3170x; 1.0003x over previous
import jax
import jax.numpy as jnp
from jax.experimental import pallas as pl
from jax.experimental.pallas import tpu as pltpu

_M = 32
_CIN = 3
_COUT = 20
_NBLK = 2
_G = 8
_SUB = 2048
_TILE = _G * _SUB


def _unet_kernel(x_ref, w1_ref, wr_ref, bn_ref, w2_ref, blin_ref, o_ref):
    xt = x_ref[...]
    x8 = jnp.concatenate(
        [xt[:, _SUB * k:_SUB * (k + 1)] for k in range(_G)], axis=0)

    bn = bn_ref[...]
    g = [bn[:, k:k + 1] for k in range(5)]
    blin = blin_ref[...]

    h = jnp.dot(w1_ref[...], x8, preferred_element_type=jnp.float32)

    for b in range(_NBLK):
        t = jnp.maximum(h + g[b], 0.0)
        t = jnp.dot(wr_ref[2 * b], t.astype(jnp.bfloat16),
                    preferred_element_type=jnp.float32)
        t = jnp.maximum(t + g[2 + b], 0.0)
        t = jnp.dot(wr_ref[2 * b + 1], t.astype(jnp.bfloat16),
                    preferred_element_type=jnp.float32)
        h = h + t

    h = jnp.maximum(h + g[4], 0.0)
    out = jnp.dot(w2_ref[...], h.astype(jnp.bfloat16),
                  preferred_element_type=jnp.float32) + blin
    ob = out.astype(jnp.bfloat16)
    for k in range(_G):
        o_ref[:, pl.ds(_SUB * k, _SUB)] = ob[24 * k:24 * k + _COUT, :]


def _expand(w):
    eye = jnp.eye(_G, dtype=w.dtype)
    r, c = w.shape
    out = (eye[:, None, :, None] * w[None, :, None, :])
    return out.reshape(_G * r, _G * c).astype(jnp.bfloat16)


def _pack_params(wsub, wres, gres, bres, gfin, bfin, wlin, blin):
    w1 = _expand(wsub.T)
    wr = []
    for b in range(_NBLK):
        g1 = gres[2 * b + 1].reshape(_M, 1)
        g0 = gres[2 * b].reshape(1, _M)
        wr.append(_expand(wres[2 * b].T * g1 * g0))
        wr.append(_expand(wres[2 * b + 1].T))
    wr = jnp.stack(wr)

    wlin_pad = jnp.zeros((24, _M), jnp.float32).at[:_COUT].set(wlin.T)
    w2 = _expand(wlin_pad * gfin.reshape(1, _M))

    tile8 = lambda v: jnp.tile(v.reshape(-1).astype(jnp.float32), (_G,))
    bn = jnp.stack([
        tile8(bres[0] / gres[0]), tile8(bres[2] / gres[2]),
        tile8(bres[1]), tile8(bres[3]),
        tile8(bfin / gfin),
    ], axis=1)
    blin_pad = jnp.zeros((24,), jnp.float32).at[:_COUT].set(blin.reshape(-1))
    blin_c = tile8(blin_pad).reshape(_G * 24, 1)
    return w1, wr, bn, w2, blin_c


def kernel(x, wsub, wres, gres, bres, gfin, bfin, wlin, blin):
    n = x.shape[0]
    n_pad = pl.cdiv(n, _TILE) * _TILE
    x_t = jnp.pad(x.astype(jnp.bfloat16), ((0, n_pad - n), (0, 0))).T

    w1, wr, bn, w2, blin_c = _pack_params(
        wsub, wres, gres, bres, gfin, bfin, wlin, blin)

    full = lambda shape: pl.BlockSpec(shape, lambda i: (0,) * len(shape))
    out_t = pl.pallas_call(
        _unet_kernel,
        out_shape=jax.ShapeDtypeStruct((_COUT, n_pad), jnp.bfloat16),
        grid=(n_pad // _TILE,),
        in_specs=[
            pl.BlockSpec((_CIN, _TILE), lambda i: (0, i)),
            full((_G * _M, _G * _CIN)),
            full((2 * _NBLK, _G * _M, _G * _M)),
            full((_G * _M, 5)),
            full((_G * 24, _G * _M)),
            full((_G * 24, 1)),
        ],
        out_specs=pl.BlockSpec((_COUT, _TILE), lambda i: (0, i)),
        compiler_params=pltpu.CompilerParams(
            dimension_semantics=("parallel",)),
    )(x_t, w1, wr, bn, w2, blin_c)

    return out_t[:, :n].T
